# baseline (speedup 1.0000x reference)
import jax
import jax.numpy as jnp
from jax import lax
from jax.experimental import pallas as pl
from jax.experimental.pallas import tpu as pltpu
from jax.experimental.pallas import tpu_sc as plsc

_NUM_WORKERS = 32
_NBUF = 8
_LAG = 4
_PAD_BLOCK = 1000


def kernel(x, table):
    batch, seq = x.shape
    vocab, emb = table.shape
    emb2 = 2 * emb

    def pad_body(t_ref, o_ref):
        v = t_ref[...]
        o_ref[...] = jnp.concatenate([v, v], axis=1)

    table_pad = pl.pallas_call(
        pad_body,
        grid=(vocab // _PAD_BLOCK,),
        in_specs=[pl.BlockSpec((_PAD_BLOCK, emb), lambda i: (i, 0))],
        out_specs=pl.BlockSpec((_PAD_BLOCK, emb2), lambda i: (i, 0)),
        out_shape=jax.ShapeDtypeStruct((vocab, emb2), table.dtype),
    )(table)

    rows_per_worker = batch // _NUM_WORKERS
    num_groups = rows_per_worker // _NBUF
    mesh = plsc.VectorSubcoreMesh(core_axis_name="c", subcore_axis_name="s")

    @pl.kernel(
        out_type=jax.ShapeDtypeStruct((batch, seq, emb), table.dtype),
        mesh=mesh,
        compiler_params=pltpu.CompilerParams(use_tc_tiling_on_sc=False),
        scratch_types=[
            pltpu.VMEM((rows_per_worker, seq), jnp.int32),
            [pltpu.VMEM((seq, emb2), table.dtype) for _ in range(_NBUF)],
            [pltpu.SemaphoreType.DMA for _ in range(_NBUF)],
            [pltpu.SemaphoreType.DMA for _ in range(_NBUF)],
        ],
    )
    def gather_kernel(tp_hbm, x_hbm, out_hbm, idx_all, rows, gsem, wsem):
        wid = lax.axis_index("s") * 2 + lax.axis_index("c")
        rbase = wid * rows_per_worker
        pltpu.sync_copy(x_hbm.at[pl.ds(rbase, rows_per_worker)], idx_all)

        def start_gather(r, b):
            pltpu.async_copy(tp_hbm.at[idx_all.at[r]], rows[b], gsem[b])

        def wait_gather(r, b):
            pltpu.make_async_copy(
                tp_hbm.at[idx_all.at[r]], rows[b], gsem[b]
            ).wait()

        def start_wb(r, b):
            pltpu.async_copy(
                rows[b].at[:, pl.ds(0, emb)], out_hbm.at[rbase + r], wsem[b]
            )

        def wait_wb(r, b):
            pltpu.make_async_copy(
                rows[b].at[:, pl.ds(0, emb)], out_hbm.at[rbase + r], wsem[b]
            ).wait()

        for i in range(_NBUF):
            start_gather(i, i)
            if i >= _LAG:
                d = i - _LAG
                wait_gather(d, d % _NBUF)
                start_wb(d, d % _NBUF)

        @pl.loop(1, num_groups)
        def _(k):
            r0 = k * _NBUF
            for i in range(_NBUF):
                r = r0 + i
                wait_wb(r - _NBUF, i)
                start_gather(r, i)
                d = r - _LAG
                bd = (i + _NBUF - _LAG) % _NBUF
                wait_gather(d, bd)
                start_wb(d, bd)

        for d in range(rows_per_worker - _LAG, rows_per_worker):
            wait_gather(d, d % _NBUF)
            start_wb(d, d % _NBUF)
        for b in range(_NBUF):
            wait_wb(rows_per_worker - _NBUF + b, b)

    return gather_kernel(table_pad, x)
